# Initial kernel scaffold; baseline (speedup 1.0000x reference)
import jax
import jax.numpy as jnp
from jax.experimental import pallas as pl


def kernel(x, edge_index, W1, b1, W2, b2):
    raise NotImplementedError("write your pallas kernel here")


# speedup vs baseline: 42.1802x; 42.1802x over previous
import functools

import jax
import jax.numpy as jnp
from jax import lax
from jax.experimental import pallas as pl
from jax.experimental.pallas import tpu as pltpu
from jax.experimental.pallas import tpu_sc as plsc

N = 100000
E = 1600000
D_IN = 128
D_HID = 16
D_OUT = 8

NC = 2
NS = 16
NW = NC * NS

BK = 128
CH = 4
MACRO = CH * BK
MACROS_PER_TILE = 98
E_PAD = NW * MACROS_PER_TILE * MACRO
ROWS_PER_TILE = MACROS_PER_TILE * CH

NACC = 100352
ROWS_ACC = NACC // NS
ZROWS = 392
BLK = 2048
GRID = NACC // BLK

_MESH = plsc.VectorSubcoreMesh(core_axis_name="c", subcore_axis_name="s")
_SC_PARAMS = pltpu.CompilerParams(use_tc_tiling_on_sc=False)


def _tile_base(unit):
    c = lax.axis_index("c")
    s = lax.axis_index("s")
    return (c * NS + s) * unit, c, s


@functools.partial(
    pl.kernel,
    out_type=jax.ShapeDtypeStruct((NC, NACC), jnp.float32),
    mesh=_MESH,
    compiler_params=_SC_PARAMS,
    scratch_types=[
        pltpu.VMEM_SHARED((NACC,), jnp.float32),
        pltpu.VMEM((ZROWS,), jnp.float32),
        pltpu.VMEM((BK,), jnp.float32),
        pltpu.VMEM((CH, BK), jnp.int32),
        pltpu.VMEM((CH, BK), jnp.int32),
        pltpu.SemaphoreType.DMA,
        pltpu.SemaphoreType.DMA,
    ],
)
def _deg_kernel(dst_hbm, deg_out, acc, zbuf, ones, d0, d1, si0, si1):
    base, c, s = _tile_base(ROWS_PER_TILE)

    def fill(i, _):
        zbuf[pl.ds(i * 16, 16)] = jnp.zeros((16,), jnp.float32)
        return 0

    lax.fori_loop(0, ZROWS // 16, fill, 0)

    def fill1(i, _):
        ones[pl.ds(i * 16, 16)] = jnp.ones((16,), jnp.float32)
        return 0

    lax.fori_loop(0, BK // 16, fill1, 0)

    r0 = s * ROWS_ACC
    for r in range(ROWS_ACC // ZROWS):
        pltpu.sync_copy(zbuf, acc.at[pl.ds(r0 + r * ZROWS, ZROWS)])
    plsc.subcore_barrier()

    dbufs = (d0, d1)
    sems = (si0, si1)

    def load(j, b):
        pltpu.async_copy(dst_hbm.at[pl.ds(base + j * CH, CH)], dbufs[b], sems[b])

    def wait(b):
        pltpu.make_async_copy(dst_hbm.at[pl.ds(0, CH)], dbufs[b], sems[b]).wait()

    def scat(b):
        for r in range(CH):
            pltpu.sync_copy(ones, acc.at[dbufs[b].at[r]], add=True)

    load(0, 0)
    load(1, 1)

    def body(i, _):
        j = 2 * i
        wait(0)
        scat(0)

        @pl.when(j + 2 < MACROS_PER_TILE)
        def _():
            load(j + 2, 0)

        wait(1)
        scat(1)

        @pl.when(j + 3 < MACROS_PER_TILE)
        def _():
            load(j + 3, 1)

        return 0

    lax.fori_loop(0, MACROS_PER_TILE // 2, body, 0)
    plsc.subcore_barrier()
    pltpu.sync_copy(acc.at[pl.ds(s * ROWS_ACC, ROWS_ACC)],
                    deg_out.at[c, pl.ds(s * ROWS_ACC, ROWS_ACC)])


@functools.partial(
    pl.kernel,
    out_type=jax.ShapeDtypeStruct((NC, NACC, D_HID), jnp.float32),
    mesh=_MESH,
    compiler_params=_SC_PARAMS,
    scratch_types=[
        pltpu.VMEM_SHARED((NACC, D_HID), jnp.float32),
        pltpu.VMEM((ZROWS, D_HID), jnp.float32),
        pltpu.VMEM((CH, BK), jnp.int32),
        pltpu.VMEM((CH, BK), jnp.int32),
        pltpu.VMEM((CH, BK), jnp.int32),
        pltpu.VMEM((CH, BK), jnp.int32),
        pltpu.VMEM((MACRO, D_HID), jnp.float32),
        pltpu.VMEM((MACRO, D_HID), jnp.float32),
        pltpu.SemaphoreType.DMA,
        pltpu.SemaphoreType.DMA,
        pltpu.SemaphoreType.DMA,
        pltpu.SemaphoreType.DMA,
    ],
)
def _agg_kernel(src_hbm, dst_hbm, table, parts, acc, zbuf,
                s0, s1, d0, d1, m0, m1, si0, si1, sg0, sg1):
    base, c, s = _tile_base(ROWS_PER_TILE)

    def fill(i, _):
        zbuf[i, :] = jnp.zeros((D_HID,), jnp.float32)
        return 0

    lax.fori_loop(0, ZROWS, fill, 0)
    r0 = s * ROWS_ACC
    for r in range(ROWS_ACC // ZROWS):
        pltpu.sync_copy(zbuf, acc.at[pl.ds(r0 + r * ZROWS, ZROWS), :])
    plsc.subcore_barrier()

    sbufs = (s0, s1)
    dbufs = (d0, d1)
    mbufs = (m0, m1)
    isems = (si0, si1)
    gsems = (sg0, sg1)

    def load_idx(j, b):
        pltpu.async_copy(src_hbm.at[pl.ds(base + j * CH, CH)], sbufs[b], isems[b])
        pltpu.async_copy(dst_hbm.at[pl.ds(base + j * CH, CH)], dbufs[b], isems[b])

    def wait_idx(b):
        pltpu.make_async_copy(src_hbm.at[pl.ds(0, CH)], sbufs[b], isems[b]).wait()
        pltpu.make_async_copy(dst_hbm.at[pl.ds(0, CH)], dbufs[b], isems[b]).wait()

    def fire_gathers(b):
        for r in range(CH):
            pltpu.async_copy(table.at[sbufs[b].at[r]],
                             mbufs[b].at[pl.ds(r * BK, BK), :], gsems[b])

    def drain_gathers(b):
        for r in range(CH):
            pltpu.make_async_copy(table.at[sbufs[b].at[r]],
                                  mbufs[b].at[pl.ds(r * BK, BK), :],
                                  gsems[b]).wait()

    def scatter(b):
        for r in range(CH):
            pltpu.sync_copy(mbufs[b].at[pl.ds(r * BK, BK), :],
                            acc.at[dbufs[b].at[r]], add=True)

    load_idx(0, 0)
    wait_idx(0)
    fire_gathers(0)
    load_idx(1, 1)

    def body(i, _):
        j = 2 * i
        wait_idx(1)
        fire_gathers(1)
        drain_gathers(0)
        scatter(0)

        @pl.when(j + 2 < MACROS_PER_TILE)
        def _():
            load_idx(j + 2, 0)
            wait_idx(0)
            fire_gathers(0)

        drain_gathers(1)
        scatter(1)

        @pl.when(j + 3 < MACROS_PER_TILE)
        def _():
            load_idx(j + 3, 1)

        return 0

    lax.fori_loop(0, MACROS_PER_TILE // 2, body, 0)
    plsc.subcore_barrier()
    pltpu.sync_copy(acc.at[pl.ds(s * ROWS_ACC, ROWS_ACC), :],
                    parts.at[c, pl.ds(s * ROWS_ACC, ROWS_ACC), :])


def _mm1_body(x_ref, w_ref, dg_ref, xwp_ref, dis_ref):
    deg = dg_ref[0, :, :] + dg_ref[1, :, :] + 1.0
    dis = lax.rsqrt(deg)
    xw = jnp.dot(x_ref[...], w_ref[...], preferred_element_type=jnp.float32)
    xwp_ref[...] = xw * dis
    dis_ref[...] = dis


def _mid_body(p_ref, xwp_ref, dis_ref, b1_ref, out_ref):
    dis = dis_ref[...]
    ssum = p_ref[0, :, :] + p_ref[1, :, :] + xwp_ref[...]
    h = jnp.maximum(ssum * dis + b1_ref[...], 0.0)
    out_ref[...] = h * dis


def _out_body(q_ref, hp_ref, dis_ref, w2_ref, b2_ref, out_ref):
    ssum = q_ref[0, :, :] + q_ref[1, :, :] + hp_ref[...]
    y = jnp.dot(ssum, w2_ref[...], preferred_element_type=jnp.float32)
    out_ref[...] = y * dis_ref[...] + b2_ref[...]


def kernel(x, edge_index, W1, b1, W2, b2):
    src = edge_index[0]
    dst = edge_index[1]
    pad = E_PAD - E
    srcp = jnp.concatenate([src, jnp.zeros((pad,), jnp.int32)]).reshape(-1, BK)
    dstp = jnp.concatenate([dst, jnp.full((pad,), N, jnp.int32)]).reshape(-1, BK)

    deg_parts = _deg_kernel(dstp)
    dg = deg_parts.reshape(NC, NACC, 1)

    xwp, dis = pl.pallas_call(
        _mm1_body,
        grid=(GRID,),
        in_specs=[
            pl.BlockSpec((BLK, D_IN), lambda i: (i, 0)),
            pl.BlockSpec((D_IN, D_HID), lambda i: (0, 0)),
            pl.BlockSpec((NC, BLK, 1), lambda i: (0, i, 0)),
        ],
        out_specs=[
            pl.BlockSpec((BLK, D_HID), lambda i: (i, 0)),
            pl.BlockSpec((BLK, 1), lambda i: (i, 0)),
        ],
        out_shape=[
            jax.ShapeDtypeStruct((NACC, D_HID), jnp.float32),
            jax.ShapeDtypeStruct((NACC, 1), jnp.float32),
        ],
    )(x, W1, dg)

    parts = _agg_kernel(srcp, dstp, xwp)

    hp = pl.pallas_call(
        _mid_body,
        grid=(GRID,),
        in_specs=[
            pl.BlockSpec((NC, BLK, D_HID), lambda i: (0, i, 0)),
            pl.BlockSpec((BLK, D_HID), lambda i: (i, 0)),
            pl.BlockSpec((BLK, 1), lambda i: (i, 0)),
            pl.BlockSpec((1, D_HID), lambda i: (0, 0)),
        ],
        out_specs=pl.BlockSpec((BLK, D_HID), lambda i: (i, 0)),
        out_shape=jax.ShapeDtypeStruct((NACC, D_HID), jnp.float32),
    )(parts, xwp, dis, b1.reshape(1, D_HID))

    qarts = _agg_kernel(srcp, dstp, hp)

    out = pl.pallas_call(
        _out_body,
        grid=(GRID,),
        in_specs=[
            pl.BlockSpec((NC, BLK, D_HID), lambda i: (0, i, 0)),
            pl.BlockSpec((BLK, D_HID), lambda i: (i, 0)),
            pl.BlockSpec((BLK, 1), lambda i: (i, 0)),
            pl.BlockSpec((D_HID, D_OUT), lambda i: (0, 0)),
            pl.BlockSpec((1, D_OUT), lambda i: (0, 0)),
        ],
        out_specs=pl.BlockSpec((BLK, D_OUT), lambda i: (i, 0)),
        out_shape=jax.ShapeDtypeStruct((NACC, D_OUT), jnp.float32),
    )(qarts, hp, dis, W2, b2.reshape(1, D_OUT))

    return out[:N]


# speedup vs baseline: 44.1852x; 1.0475x over previous
import functools

import jax
import jax.numpy as jnp
from jax import lax
from jax.experimental import pallas as pl
from jax.experimental.pallas import tpu as pltpu
from jax.experimental.pallas import tpu_sc as plsc

N = 100000
E = 1600000
D_IN = 128
D_HID = 16
D_OUT = 8

NC = 2
NS = 16
NW = NC * NS

BK = 128
CH = 4
MACRO = CH * BK
MACROS_PER_TILE = 98
E_PAD = NW * MACROS_PER_TILE * MACRO
ROWS_PER_TILE = MACROS_PER_TILE * CH

NACC = 100352
ROWS_ACC = NACC // NS
ZROWS = 392
BLK = 2048
GRID = NACC // BLK

_MESH = plsc.VectorSubcoreMesh(core_axis_name="c", subcore_axis_name="s")
_SC_PARAMS = pltpu.CompilerParams(use_tc_tiling_on_sc=False)


def _tile_base(unit):
    c = lax.axis_index("c")
    s = lax.axis_index("s")
    return (c * NS + s) * unit, c, s


@functools.partial(
    pl.kernel,
    out_type=jax.ShapeDtypeStruct((NC, NACC), jnp.float32),
    mesh=_MESH,
    compiler_params=_SC_PARAMS,
    scratch_types=[
        pltpu.VMEM_SHARED((NACC,), jnp.float32),
        pltpu.VMEM((ZROWS,), jnp.float32),
        pltpu.VMEM((BK,), jnp.float32),
        pltpu.VMEM((CH, BK), jnp.int32),
        pltpu.VMEM((CH, BK), jnp.int32),
        pltpu.SemaphoreType.DMA,
        pltpu.SemaphoreType.DMA,
        pltpu.SemaphoreType.DMA,
        pltpu.SemaphoreType.DMA,
    ],
)
def _deg_kernel(dst_hbm, deg_out, acc, zbuf, ones, d0, d1, si0, si1, ss0, ss1):
    base, c, s = _tile_base(ROWS_PER_TILE)

    def fill(i, _):
        zbuf[pl.ds(i * 16, 16)] = jnp.zeros((16,), jnp.float32)
        return 0

    lax.fori_loop(0, ZROWS // 16, fill, 0)

    def fill1(i, _):
        ones[pl.ds(i * 16, 16)] = jnp.ones((16,), jnp.float32)
        return 0

    lax.fori_loop(0, BK // 16, fill1, 0)

    r0 = s * ROWS_ACC
    for r in range(ROWS_ACC // ZROWS):
        pltpu.sync_copy(zbuf, acc.at[pl.ds(r0 + r * ZROWS, ZROWS)])
    plsc.subcore_barrier()

    dbufs = (d0, d1)
    sems = (si0, si1)
    ssems = (ss0, ss1)

    def load(j, b):
        pltpu.async_copy(dst_hbm.at[pl.ds(base + j * CH, CH)], dbufs[b], sems[b])

    def wait(b):
        pltpu.make_async_copy(dst_hbm.at[pl.ds(0, CH)], dbufs[b], sems[b]).wait()

    def scat(b):
        for r in range(CH):
            pltpu.async_copy(ones, acc.at[dbufs[b].at[r]], ssems[b], add=True)
        for r in range(CH):
            pltpu.make_async_copy(ones, acc.at[dbufs[b].at[r]], ssems[b]).wait()

    load(0, 0)
    load(1, 1)

    def body(i, _):
        j = 2 * i
        wait(0)
        scat(0)

        @pl.when(j + 2 < MACROS_PER_TILE)
        def _():
            load(j + 2, 0)

        wait(1)
        scat(1)

        @pl.when(j + 3 < MACROS_PER_TILE)
        def _():
            load(j + 3, 1)

        return 0

    lax.fori_loop(0, MACROS_PER_TILE // 2, body, 0)
    plsc.subcore_barrier()
    pltpu.sync_copy(acc.at[pl.ds(s * ROWS_ACC, ROWS_ACC)],
                    deg_out.at[c, pl.ds(s * ROWS_ACC, ROWS_ACC)])


@functools.partial(
    pl.kernel,
    out_type=jax.ShapeDtypeStruct((NC, NACC, D_HID), jnp.float32),
    mesh=_MESH,
    compiler_params=_SC_PARAMS,
    scratch_types=[
        pltpu.VMEM_SHARED((NACC, D_HID), jnp.float32),
        pltpu.VMEM((ZROWS, D_HID), jnp.float32),
        pltpu.VMEM((CH, BK), jnp.int32),
        pltpu.VMEM((CH, BK), jnp.int32),
        pltpu.VMEM((CH, BK), jnp.int32),
        pltpu.VMEM((CH, BK), jnp.int32),
        pltpu.VMEM((MACRO, D_HID), jnp.float32),
        pltpu.VMEM((MACRO, D_HID), jnp.float32),
        pltpu.VMEM((CH, BK), jnp.int32),
        pltpu.VMEM((CH, BK), jnp.int32),
        pltpu.SemaphoreType.DMA,
        pltpu.SemaphoreType.DMA,
        pltpu.SemaphoreType.DMA,
        pltpu.SemaphoreType.DMA,
        pltpu.SemaphoreType.DMA,
        pltpu.SemaphoreType.DMA,
    ],
)
def _agg_kernel(src_hbm, dst_hbm, table, parts, acc, zbuf,
                s0, s1, d0, d1, m0, m1, c0, c1,
                si0, si1, sg0, sg1, ss0, ss1):
    base, c, s = _tile_base(ROWS_PER_TILE)

    def fill(i, _):
        zbuf[i, :] = jnp.zeros((D_HID,), jnp.float32)
        return 0

    lax.fori_loop(0, ZROWS, fill, 0)
    r0 = s * ROWS_ACC
    for r in range(ROWS_ACC // ZROWS):
        pltpu.sync_copy(zbuf, acc.at[pl.ds(r0 + r * ZROWS, ZROWS), :])
    plsc.subcore_barrier()

    sbufs = (s0, s1)
    dbufs = (d0, d1)
    mbufs = (m0, m1)
    cbufs = (c0, c1)
    isems = (si0, si1)
    gsems = (sg0, sg1)
    ssems = (ss0, ss1)

    def load_idx(j, b):
        pltpu.async_copy(src_hbm.at[pl.ds(base + j * CH, CH)], sbufs[b], isems[b])
        pltpu.async_copy(dst_hbm.at[pl.ds(base + j * CH, CH)], dbufs[b], isems[b])

    def wait_idx(b):
        pltpu.make_async_copy(src_hbm.at[pl.ds(0, CH)], sbufs[b], isems[b]).wait()
        pltpu.make_async_copy(dst_hbm.at[pl.ds(0, CH)], dbufs[b], isems[b]).wait()

    def fire_gathers(b):
        for r in range(CH):
            pltpu.async_copy(table.at[sbufs[b].at[r]],
                             mbufs[b].at[pl.ds(r * BK, BK), :], gsems[b])

    def drain_gathers(b):
        for r in range(CH):
            pltpu.make_async_copy(table.at[sbufs[b].at[r]],
                                  mbufs[b].at[pl.ds(r * BK, BK), :],
                                  gsems[b]).wait()

    def fire_scatters(b):
        for r in range(CH):
            for k in range(BK // 16):
                cbufs[b][r, pl.ds(k * 16, 16)] = dbufs[b][r, pl.ds(k * 16, 16)]
        for r in range(CH):
            pltpu.async_copy(mbufs[b].at[pl.ds(r * BK, BK), :],
                             acc.at[cbufs[b].at[r]], ssems[b], add=True)

    def drain_scatters(b):
        for r in range(CH):
            pltpu.make_async_copy(mbufs[b].at[pl.ds(r * BK, BK), :],
                                  acc.at[cbufs[b].at[r]], ssems[b]).wait()

    load_idx(0, 0)
    wait_idx(0)
    fire_gathers(0)
    load_idx(1, 1)

    def body(i, _):
        j = 2 * i

        @pl.when(j > 0)
        def _():
            drain_scatters(1)

        wait_idx(1)
        fire_gathers(1)
        drain_gathers(0)
        fire_scatters(0)

        @pl.when(j + 2 < MACROS_PER_TILE)
        def _():
            load_idx(j + 2, 0)
            wait_idx(0)

        drain_gathers(1)
        fire_scatters(1)
        drain_scatters(0)

        @pl.when(j + 2 < MACROS_PER_TILE)
        def _():
            fire_gathers(0)

        @pl.when(j + 3 < MACROS_PER_TILE)
        def _():
            load_idx(j + 3, 1)

        return 0

    lax.fori_loop(0, MACROS_PER_TILE // 2, body, 0)
    drain_scatters(1)
    plsc.subcore_barrier()
    pltpu.sync_copy(acc.at[pl.ds(s * ROWS_ACC, ROWS_ACC), :],
                    parts.at[c, pl.ds(s * ROWS_ACC, ROWS_ACC), :])


def _mm1_body(x_ref, w_ref, dg_ref, xwp_ref, dis_ref):
    deg = dg_ref[0, :, :] + dg_ref[1, :, :] + 1.0
    dis = lax.rsqrt(deg)
    xw = jnp.dot(x_ref[...], w_ref[...], preferred_element_type=jnp.float32)
    xwp_ref[...] = xw * dis
    dis_ref[...] = dis


def _mid_body(p_ref, xwp_ref, dis_ref, b1_ref, out_ref):
    dis = dis_ref[...]
    ssum = p_ref[0, :, :] + p_ref[1, :, :] + xwp_ref[...]
    h = jnp.maximum(ssum * dis + b1_ref[...], 0.0)
    out_ref[...] = h * dis


def _out_body(q_ref, hp_ref, dis_ref, w2_ref, b2_ref, out_ref):
    ssum = q_ref[0, :, :] + q_ref[1, :, :] + hp_ref[...]
    y = jnp.dot(ssum, w2_ref[...], preferred_element_type=jnp.float32)
    out_ref[...] = y * dis_ref[...] + b2_ref[...]


def kernel(x, edge_index, W1, b1, W2, b2):
    src = edge_index[0]
    dst = edge_index[1]
    pad = E_PAD - E
    srcp = jnp.concatenate([src, jnp.zeros((pad,), jnp.int32)]).reshape(-1, BK)
    dstp = jnp.concatenate([dst, jnp.full((pad,), N, jnp.int32)]).reshape(-1, BK)

    deg_parts = _deg_kernel(dstp)
    dg = deg_parts.reshape(NC, NACC, 1)

    xwp, dis = pl.pallas_call(
        _mm1_body,
        grid=(GRID,),
        in_specs=[
            pl.BlockSpec((BLK, D_IN), lambda i: (i, 0)),
            pl.BlockSpec((D_IN, D_HID), lambda i: (0, 0)),
            pl.BlockSpec((NC, BLK, 1), lambda i: (0, i, 0)),
        ],
        out_specs=[
            pl.BlockSpec((BLK, D_HID), lambda i: (i, 0)),
            pl.BlockSpec((BLK, 1), lambda i: (i, 0)),
        ],
        out_shape=[
            jax.ShapeDtypeStruct((NACC, D_HID), jnp.float32),
            jax.ShapeDtypeStruct((NACC, 1), jnp.float32),
        ],
    )(x, W1, dg)

    parts = _agg_kernel(srcp, dstp, xwp)

    hp = pl.pallas_call(
        _mid_body,
        grid=(GRID,),
        in_specs=[
            pl.BlockSpec((NC, BLK, D_HID), lambda i: (0, i, 0)),
            pl.BlockSpec((BLK, D_HID), lambda i: (i, 0)),
            pl.BlockSpec((BLK, 1), lambda i: (i, 0)),
            pl.BlockSpec((1, D_HID), lambda i: (0, 0)),
        ],
        out_specs=pl.BlockSpec((BLK, D_HID), lambda i: (i, 0)),
        out_shape=jax.ShapeDtypeStruct((NACC, D_HID), jnp.float32),
    )(parts, xwp, dis, b1.reshape(1, D_HID))

    qarts = _agg_kernel(srcp, dstp, hp)

    out = pl.pallas_call(
        _out_body,
        grid=(GRID,),
        in_specs=[
            pl.BlockSpec((NC, BLK, D_HID), lambda i: (0, i, 0)),
            pl.BlockSpec((BLK, D_HID), lambda i: (i, 0)),
            pl.BlockSpec((BLK, 1), lambda i: (i, 0)),
            pl.BlockSpec((D_HID, D_OUT), lambda i: (0, 0)),
            pl.BlockSpec((1, D_OUT), lambda i: (0, 0)),
        ],
        out_specs=pl.BlockSpec((BLK, D_OUT), lambda i: (i, 0)),
        out_shape=jax.ShapeDtypeStruct((NACC, D_OUT), jnp.float32),
    )(qarts, hp, dis, W2, b2.reshape(1, D_OUT))

    return out[:N]


# speedup vs baseline: 80.5173x; 1.8223x over previous
import functools

import jax
import jax.numpy as jnp
import numpy as np
from jax import lax
from jax.experimental import pallas as pl
from jax.experimental.pallas import tpu as pltpu
from jax.experimental.pallas import tpu_sc as plsc

N = 100000
E = 1600000
D_IN = 128
D_HID = 16
D_OUT = 8

NC = 2
NS = 16
NW = NC * NS

BK = 128
CH = 4
MACRO = CH * BK
EPT = E // NW
FULL_MACROS = 96
TAIL0 = FULL_MACROS * MACRO + MACRO
TAIL1 = TAIL0 + 2 * BK
TBK = 80

NACC = 100352
ROWS_ACC = NACC // NS
ZROWS = 392
PR = NACC // 8
BLK = 256
GRID = PR // BLK

_MESH = plsc.VectorSubcoreMesh(core_axis_name="c", subcore_axis_name="s")
_SC_PARAMS = pltpu.CompilerParams(use_tc_tiling_on_sc=False)


def _tile_base():
    c = lax.axis_index("c")
    s = lax.axis_index("s")
    return (c * NS + s) * EPT, c, s


def _zero_acc(acc, zbuf, s, width):
    def fill(i, _):
        if width == 1:
            zbuf[pl.ds(i * 16, 16)] = jnp.zeros((16,), jnp.float32)
        else:
            zbuf[i, :] = jnp.zeros((width,), jnp.float32)
        return 0

    lax.fori_loop(0, ZROWS if width > 1 else ZROWS // 16, fill, 0)
    r0 = s * ROWS_ACC
    for r in range(ROWS_ACC // ZROWS):
        pltpu.sync_copy(zbuf, acc.at[pl.ds(r0 + r * ZROWS, ZROWS)])


def _copy_rows(dst2d, src1d, rows, cols):
    for r in range(rows):
        for k in range(cols // 16):
            dst2d[r, pl.ds(k * 16, 16)] = src1d[pl.ds(r * cols + k * 16, 16)]


@functools.partial(
    pl.kernel,
    out_type=jax.ShapeDtypeStruct((NC, NACC, D_HID), jnp.float32),
    mesh=_MESH,
    compiler_params=_SC_PARAMS,
    scratch_types=[
        pltpu.VMEM_SHARED((NACC, D_HID), jnp.float32),
        pltpu.VMEM((ZROWS, D_HID), jnp.float32),
        pltpu.VMEM((MACRO, D_HID), jnp.float32),
        pltpu.VMEM((MACRO,), jnp.int32),
        pltpu.VMEM((MACRO,), jnp.int32),
        pltpu.VMEM((CH, BK), jnp.int32),
        pltpu.VMEM((CH, BK), jnp.int32),
        pltpu.VMEM((TBK,), jnp.int32),
        pltpu.SemaphoreType.DMA,
        pltpu.SemaphoreType.DMA,
        pltpu.SemaphoreType.DMA,
        pltpu.SemaphoreType.DMA,
    ],
)
def _deg_kernel(dst_hbm, deg_out, acc, zbuf, ones, d0, d1, c0, c1, ct,
                si0, si1, ss0, ss1):
    base, c, s = _tile_base()
    _zero_acc(acc, zbuf, s, D_HID)

    def fill1(i, _):
        ones[i, :] = jnp.ones((D_HID,), jnp.float32)
        return 0

    lax.fori_loop(0, MACRO, fill1, 0)
    plsc.subcore_barrier()

    dbufs = (d0, d1)
    cbufs = (c0, c1)
    isems = (si0, si1)
    ssems = (ss0, ss1)

    def load_idx(j, b):
        pltpu.async_copy(dst_hbm.at[pl.ds(base + j * MACRO, MACRO)],
                         dbufs[b], isems[b])

    def wait_idx(b):
        pltpu.make_async_copy(dst_hbm.at[pl.ds(0, MACRO)], dbufs[b],
                              isems[b]).wait()

    def fire_scatters(b):
        _copy_rows(cbufs[b], dbufs[b], CH, BK)
        for r in range(CH):
            pltpu.async_copy(ones.at[pl.ds(r * BK, BK), :],
                             acc.at[cbufs[b].at[r]], ssems[b], add=True)

    def drain_scatters(b):
        for r in range(CH):
            pltpu.make_async_copy(ones.at[pl.ds(r * BK, BK), :],
                                  acc.at[cbufs[b].at[r]], ssems[b]).wait()

    load_idx(0, 0)
    load_idx(1, 1)

    def body(i, _):
        j = 2 * i
        wait_idx(0)
        fire_scatters(0)
        load_idx(j + 2, 0)
        wait_idx(1)
        fire_scatters(1)
        drain_scatters(0)

        @pl.when(j + 3 < FULL_MACROS + 1)
        def _():
            load_idx(j + 3, 1)

        drain_scatters(1)
        return 0

    lax.fori_loop(0, FULL_MACROS // 2, body, 0)
    wait_idx(0)
    fire_scatters(0)
    drain_scatters(0)
    pltpu.sync_copy(dst_hbm.at[pl.ds(base + TAIL0, 2 * BK)],
                    d1.at[pl.ds(0, 2 * BK)])
    _copy_rows(c1, d1, 2, BK)
    for r in range(2):
        pltpu.async_copy(ones.at[pl.ds(r * BK, BK), :],
                         acc.at[c1.at[r]], ss1, add=True)
    pltpu.sync_copy(dst_hbm.at[pl.ds(base + TAIL1, TBK)], ct)
    pltpu.async_copy(ones.at[pl.ds(0, TBK), :], acc.at[ct], ss1, add=True)
    for r in range(2):
        pltpu.make_async_copy(ones.at[pl.ds(r * BK, BK), :],
                              acc.at[c1.at[r]], ss1).wait()
    pltpu.make_async_copy(ones.at[pl.ds(0, TBK), :], acc.at[ct], ss1).wait()

    plsc.subcore_barrier()
    pltpu.sync_copy(acc.at[pl.ds(s * ROWS_ACC, ROWS_ACC), :],
                    deg_out.at[c, pl.ds(s * ROWS_ACC, ROWS_ACC), :])


@functools.partial(
    pl.kernel,
    out_type=jax.ShapeDtypeStruct((NC, NACC, D_HID), jnp.float32),
    mesh=_MESH,
    compiler_params=_SC_PARAMS,
    scratch_types=[
        pltpu.VMEM_SHARED((NACC, D_HID), jnp.float32),
        pltpu.VMEM((ZROWS, D_HID), jnp.float32),
        pltpu.VMEM((MACRO,), jnp.int32),
        pltpu.VMEM((MACRO,), jnp.int32),
        pltpu.VMEM((MACRO,), jnp.int32),
        pltpu.VMEM((MACRO,), jnp.int32),
        pltpu.VMEM((MACRO, D_HID), jnp.float32),
        pltpu.VMEM((MACRO, D_HID), jnp.float32),
        pltpu.VMEM((CH, BK), jnp.int32),
        pltpu.VMEM((CH, BK), jnp.int32),
        pltpu.VMEM((TBK,), jnp.int32),
        pltpu.SemaphoreType.DMA,
        pltpu.SemaphoreType.DMA,
        pltpu.SemaphoreType.DMA,
        pltpu.SemaphoreType.DMA,
        pltpu.SemaphoreType.DMA,
        pltpu.SemaphoreType.DMA,
    ],
)
def _agg_kernel(src_hbm, dst_hbm, table, parts, acc, zbuf,
                s0, s1, d0, d1, m0, m1, c0, c1, ct,
                si0, si1, sg0, sg1, ss0, ss1):
    base, c, s = _tile_base()
    _zero_acc(acc, zbuf, s, D_HID)
    plsc.subcore_barrier()

    sbufs = (s0, s1)
    dbufs = (d0, d1)
    mbufs = (m0, m1)
    cbufs = (c0, c1)
    isems = (si0, si1)
    gsems = (sg0, sg1)
    ssems = (ss0, ss1)

    def load_idx(j, b):
        pltpu.async_copy(src_hbm.at[pl.ds(base + j * MACRO, MACRO)],
                         sbufs[b], isems[b])
        pltpu.async_copy(dst_hbm.at[pl.ds(base + j * MACRO, MACRO)],
                         dbufs[b], isems[b])

    def wait_idx(b):
        pltpu.make_async_copy(src_hbm.at[pl.ds(0, MACRO)], sbufs[b],
                              isems[b]).wait()
        pltpu.make_async_copy(dst_hbm.at[pl.ds(0, MACRO)], dbufs[b],
                              isems[b]).wait()

    def fire_gathers(b):
        for r in range(CH):
            pltpu.async_copy(table.at[sbufs[b].at[pl.ds(r * BK, BK)]],
                             mbufs[b].at[pl.ds(r * BK, BK), :], gsems[b])

    def drain_gathers(b):
        for r in range(CH):
            pltpu.make_async_copy(table.at[sbufs[b].at[pl.ds(r * BK, BK)]],
                                  mbufs[b].at[pl.ds(r * BK, BK), :],
                                  gsems[b]).wait()

    def fire_scatters(b):
        _copy_rows(cbufs[b], dbufs[b], CH, BK)
        for r in range(CH):
            pltpu.async_copy(mbufs[b].at[pl.ds(r * BK, BK), :],
                             acc.at[cbufs[b].at[r]], ssems[b], add=True)

    def drain_scatters(b):
        for r in range(CH):
            pltpu.make_async_copy(mbufs[b].at[pl.ds(r * BK, BK), :],
                                  acc.at[cbufs[b].at[r]], ssems[b]).wait()

    load_idx(0, 0)
    wait_idx(0)
    fire_gathers(0)
    load_idx(1, 1)

    def body(i, _):
        j = 2 * i

        @pl.when(j > 0)
        def _():
            drain_scatters(1)

        wait_idx(1)
        fire_gathers(1)
        drain_gathers(0)
        fire_scatters(0)
        load_idx(j + 2, 0)
        wait_idx(0)
        drain_gathers(1)
        fire_scatters(1)
        drain_scatters(0)
        fire_gathers(0)

        @pl.when(j + 3 < FULL_MACROS + 1)
        def _():
            load_idx(j + 3, 1)

        return 0

    lax.fori_loop(0, FULL_MACROS // 2, body, 0)
    drain_scatters(1)
    drain_gathers(0)
    fire_scatters(0)
    drain_scatters(0)
    pltpu.sync_copy(src_hbm.at[pl.ds(base + TAIL0, 2 * BK)],
                    s1.at[pl.ds(0, 2 * BK)])
    pltpu.sync_copy(dst_hbm.at[pl.ds(base + TAIL0, 2 * BK)],
                    d1.at[pl.ds(0, 2 * BK)])
    for r in range(2):
        pltpu.async_copy(table.at[s1.at[pl.ds(r * BK, BK)]],
                         m1.at[pl.ds(r * BK, BK), :], sg1)
    _copy_rows(c1, d1, 2, BK)
    for r in range(2):
        pltpu.make_async_copy(table.at[s1.at[pl.ds(r * BK, BK)]],
                              m1.at[pl.ds(r * BK, BK), :], sg1).wait()
        pltpu.async_copy(m1.at[pl.ds(r * BK, BK), :],
                         acc.at[c1.at[r]], ss1, add=True)
    pltpu.sync_copy(src_hbm.at[pl.ds(base + TAIL1, TBK)],
                    s0.at[pl.ds(0, TBK)])
    pltpu.sync_copy(dst_hbm.at[pl.ds(base + TAIL1, TBK)], ct)
    pltpu.async_copy(table.at[s0.at[pl.ds(0, TBK)]],
                     m0.at[pl.ds(0, TBK), :], sg0)
    pltpu.make_async_copy(table.at[s0.at[pl.ds(0, TBK)]],
                          m0.at[pl.ds(0, TBK), :], sg0).wait()
    pltpu.async_copy(m0.at[pl.ds(0, TBK), :], acc.at[ct], ss1, add=True)
    for r in range(2):
        pltpu.make_async_copy(m1.at[pl.ds(r * BK, BK), :],
                              acc.at[c1.at[r]], ss1).wait()
    pltpu.make_async_copy(m0.at[pl.ds(0, TBK), :], acc.at[ct], ss1).wait()

    plsc.subcore_barrier()
    pltpu.sync_copy(acc.at[pl.ds(s * ROWS_ACC, ROWS_ACC), :],
                    parts.at[c, pl.ds(s * ROWS_ACC, ROWS_ACC), :])


def _k1_body(x_ref, w_ref, dg_ref, xwp_ref, dis_ref):
    dis = lax.rsqrt(dg_ref[0, :, :] + dg_ref[1, :, :] + 1.0)
    xw = jnp.dot(x_ref[...], w_ref[...], preferred_element_type=jnp.float32)
    xwp_ref[...] = xw * dis
    dis_ref[...] = dis


def _k2_body(p_ref, xwp_ref, dis_ref, b1_ref, out_ref):
    dis = dis_ref[...]
    ssum = p_ref[0, :, :] + p_ref[1, :, :] + xwp_ref[...]
    h = jnp.maximum(ssum * dis + b1_ref[...], 0.0)
    out_ref[...] = h * dis


def _k3_body(q_ref, hp_ref, dis_ref, w2_ref, b2_ref, out_ref):
    ssum = (q_ref[0, :, :] + q_ref[1, :, :] + hp_ref[...]) * dis_ref[...]
    y = jnp.dot(ssum, w2_ref[...], preferred_element_type=jnp.float32)
    out_ref[...] = y + b2_ref[...]


def kernel(x, edge_index, W1, b1, W2, b2):
    src = edge_index[0]
    dst = edge_index[1]
    xp = x.reshape(N // 8, 8 * D_IN)
    w1k = jnp.kron(jnp.eye(8, dtype=jnp.float32), W1)
    w2k = jnp.kron(jnp.eye(8, dtype=jnp.float32), W2)
    b1t = jnp.tile(b1, 8).reshape(1, 8 * D_HID)
    b2t = jnp.tile(b2, 8).reshape(1, 8 * D_OUT)

    deg_parts = _deg_kernel(dst)
    dg = deg_parts.reshape(NC, PR, 128)

    xwp, dis = pl.pallas_call(
        _k1_body,
        grid=(GRID,),
        in_specs=[
            pl.BlockSpec((BLK, 8 * D_IN), lambda i: (i, 0)),
            pl.BlockSpec((8 * D_IN, 128), lambda i: (0, 0)),
            pl.BlockSpec((NC, BLK, 128), lambda i: (0, i, 0)),
        ],
        out_specs=[
            pl.BlockSpec((BLK, 128), lambda i: (i, 0)),
            pl.BlockSpec((BLK, 128), lambda i: (i, 0)),
        ],
        out_shape=[
            jax.ShapeDtypeStruct((PR, 128), jnp.float32),
            jax.ShapeDtypeStruct((PR, 128), jnp.float32),
        ],
    )(xp, w1k, dg)

    parts = _agg_kernel(src, dst, xwp.reshape(NACC, D_HID))

    hp = pl.pallas_call(
        _k2_body,
        grid=(GRID,),
        in_specs=[
            pl.BlockSpec((NC, BLK, 128), lambda i: (0, i, 0)),
            pl.BlockSpec((BLK, 128), lambda i: (i, 0)),
            pl.BlockSpec((BLK, 128), lambda i: (i, 0)),
            pl.BlockSpec((1, 128), lambda i: (0, 0)),
        ],
        out_specs=pl.BlockSpec((BLK, 128), lambda i: (i, 0)),
        out_shape=jax.ShapeDtypeStruct((PR, 128), jnp.float32),
    )(parts.reshape(NC, PR, 128), xwp, dis, b1t)

    qarts = _agg_kernel(src, dst, hp.reshape(NACC, D_HID))

    out = pl.pallas_call(
        _k3_body,
        grid=(GRID,),
        in_specs=[
            pl.BlockSpec((NC, BLK, 128), lambda i: (0, i, 0)),
            pl.BlockSpec((BLK, 128), lambda i: (i, 0)),
            pl.BlockSpec((BLK, 128), lambda i: (i, 0)),
            pl.BlockSpec((128, 64), lambda i: (0, 0)),
            pl.BlockSpec((1, 64), lambda i: (0, 0)),
        ],
        out_specs=pl.BlockSpec((BLK, 64), lambda i: (i, 0)),
        out_shape=jax.ShapeDtypeStruct((PR, 64), jnp.float32),
    )(qarts.reshape(NC, PR, 128), hp, dis, w2k, b2t)

    return out.reshape(NACC, D_OUT)[:N]


# speedup vs baseline: 86.1447x; 1.0699x over previous
import functools

import jax
import jax.numpy as jnp
from jax import lax
from jax.experimental import pallas as pl
from jax.experimental.pallas import tpu as pltpu
from jax.experimental.pallas import tpu_sc as plsc

N = 100000
E = 1600000
D_IN = 128
D_HID = 16
D_OUT = 8

NC = 2
NS = 16
NW = NC * NS

BK = 128
CH = 4
MACRO = CH * BK
EPT = E // NW
FULL_MACROS = 96
TAIL0 = FULL_MACROS * MACRO + MACRO
TAIL1 = TAIL0 + 2 * BK
TBK = 80

NACC = 100352
ROWS_ACC = NACC // NS
ZROWS = 392
PR = NACC // 8
BLK = 256
GRID = PR // BLK

_MESH = plsc.VectorSubcoreMesh(core_axis_name="c", subcore_axis_name="s")
_SC_PARAMS = pltpu.CompilerParams(use_tc_tiling_on_sc=False)


def _tile_base():
    c = lax.axis_index("c")
    s = lax.axis_index("s")
    return (c * NS + s) * EPT, c, s


def _zero_acc(acc, zbuf, s, width):
    def fill(i, _):
        if width == 1:
            zbuf[pl.ds(i * 16, 16)] = jnp.zeros((16,), jnp.float32)
        else:
            zbuf[i, :] = jnp.zeros((width,), jnp.float32)
        return 0

    lax.fori_loop(0, ZROWS if width > 1 else ZROWS // 16, fill, 0)
    r0 = s * ROWS_ACC
    for r in range(ROWS_ACC // ZROWS):
        pltpu.sync_copy(zbuf, acc.at[pl.ds(r0 + r * ZROWS, ZROWS)])


def _copy_rows(dst2d, src1d, rows, cols):
    for r in range(rows):
        for k in range(cols // 16):
            dst2d[r, pl.ds(k * 16, 16)] = src1d[pl.ds(r * cols + k * 16, 16)]


@functools.partial(
    pl.kernel,
    out_type=jax.ShapeDtypeStruct((NC, NACC, D_HID), jnp.float32),
    mesh=_MESH,
    compiler_params=_SC_PARAMS,
    scratch_types=[
        pltpu.VMEM_SHARED((NACC, D_HID), jnp.float32),
        pltpu.VMEM((ZROWS, D_HID), jnp.float32),
        pltpu.VMEM((MACRO, D_HID), jnp.float32),
        pltpu.VMEM((MACRO,), jnp.int32),
        pltpu.VMEM((MACRO,), jnp.int32),
        pltpu.VMEM((CH, BK), jnp.int32),
        pltpu.VMEM((CH, BK), jnp.int32),
        pltpu.VMEM((TBK,), jnp.int32),
        pltpu.SemaphoreType.DMA,
        pltpu.SemaphoreType.DMA,
        pltpu.SemaphoreType.DMA,
        pltpu.SemaphoreType.DMA,
    ],
)
def _deg_kernel(ei_hbm, deg_out, acc, zbuf, ones, d0, d1, c0, c1, ct,
                si0, si1, ss0, ss1):
    base, c, s = _tile_base()
    _zero_acc(acc, zbuf, s, D_HID)

    def fill1(i, _):
        ones[i, :] = jnp.ones((D_HID,), jnp.float32)
        return 0

    lax.fori_loop(0, MACRO, fill1, 0)
    plsc.subcore_barrier()

    dbufs = (d0, d1)
    cbufs = (c0, c1)
    isems = (si0, si1)
    ssems = (ss0, ss1)

    def load_idx(j, b):
        pltpu.async_copy(ei_hbm.at[1, pl.ds(base + j * MACRO, MACRO)],
                         dbufs[b], isems[b])

    def wait_idx(b):
        pltpu.make_async_copy(ei_hbm.at[1, pl.ds(0, MACRO)], dbufs[b],
                              isems[b]).wait()

    def fire_scatters(b):
        _copy_rows(cbufs[b], dbufs[b], CH, BK)
        for r in range(CH):
            pltpu.async_copy(ones.at[pl.ds(r * BK, BK), :],
                             acc.at[cbufs[b].at[r]], ssems[b], add=True)

    def drain_scatters(b):
        for r in range(CH):
            pltpu.make_async_copy(ones.at[pl.ds(r * BK, BK), :],
                                  acc.at[cbufs[b].at[r]], ssems[b]).wait()

    load_idx(0, 0)
    load_idx(1, 1)

    def body(i, _):
        j = 2 * i
        wait_idx(0)
        fire_scatters(0)
        load_idx(j + 2, 0)
        wait_idx(1)
        fire_scatters(1)
        drain_scatters(0)

        @pl.when(j + 3 < FULL_MACROS + 1)
        def _():
            load_idx(j + 3, 1)

        drain_scatters(1)
        return 0

    lax.fori_loop(0, FULL_MACROS // 2, body, 0)
    wait_idx(0)
    fire_scatters(0)
    drain_scatters(0)
    pltpu.sync_copy(ei_hbm.at[1, pl.ds(base + TAIL0, 2 * BK)],
                    d1.at[pl.ds(0, 2 * BK)])
    _copy_rows(c1, d1, 2, BK)
    for r in range(2):
        pltpu.async_copy(ones.at[pl.ds(r * BK, BK), :],
                         acc.at[c1.at[r]], ss1, add=True)
    pltpu.sync_copy(ei_hbm.at[1, pl.ds(base + TAIL1, TBK)], ct)
    pltpu.async_copy(ones.at[pl.ds(0, TBK), :], acc.at[ct], ss1, add=True)
    for r in range(2):
        pltpu.make_async_copy(ones.at[pl.ds(r * BK, BK), :],
                              acc.at[c1.at[r]], ss1).wait()
    pltpu.make_async_copy(ones.at[pl.ds(0, TBK), :], acc.at[ct], ss1).wait()

    plsc.subcore_barrier()
    pltpu.sync_copy(acc.at[pl.ds(s * ROWS_ACC, ROWS_ACC), :],
                    deg_out.at[c, pl.ds(s * ROWS_ACC, ROWS_ACC), :])


@functools.partial(
    pl.kernel,
    out_type=jax.ShapeDtypeStruct((NC, NACC, D_HID), jnp.float32),
    mesh=_MESH,
    compiler_params=_SC_PARAMS,
    scratch_types=[
        pltpu.VMEM_SHARED((NACC, D_HID), jnp.float32),
        pltpu.VMEM((ZROWS, D_HID), jnp.float32),
        pltpu.VMEM((MACRO,), jnp.int32),
        pltpu.VMEM((MACRO,), jnp.int32),
        pltpu.VMEM((MACRO,), jnp.int32),
        pltpu.VMEM((MACRO,), jnp.int32),
        pltpu.VMEM((MACRO, D_HID), jnp.float32),
        pltpu.VMEM((MACRO, D_HID), jnp.float32),
        pltpu.VMEM((CH, BK), jnp.int32),
        pltpu.VMEM((CH, BK), jnp.int32),
        pltpu.VMEM((TBK,), jnp.int32),
        pltpu.SemaphoreType.DMA,
        pltpu.SemaphoreType.DMA,
        pltpu.SemaphoreType.DMA,
        pltpu.SemaphoreType.DMA,
        pltpu.SemaphoreType.DMA,
        pltpu.SemaphoreType.DMA,
    ],
)
def _agg_kernel(ei_hbm, table, parts, acc, zbuf,
                s0, s1, d0, d1, m0, m1, c0, c1, ct,
                si0, si1, sg0, sg1, ss0, ss1):
    base, c, s = _tile_base()
    _zero_acc(acc, zbuf, s, D_HID)
    plsc.subcore_barrier()

    sbufs = (s0, s1)
    dbufs = (d0, d1)
    mbufs = (m0, m1)
    cbufs = (c0, c1)
    isems = (si0, si1)
    gsems = (sg0, sg1)
    ssems = (ss0, ss1)

    def load_idx(j, b):
        pltpu.async_copy(ei_hbm.at[0, pl.ds(base + j * MACRO, MACRO)],
                         sbufs[b], isems[b])
        pltpu.async_copy(ei_hbm.at[1, pl.ds(base + j * MACRO, MACRO)],
                         dbufs[b], isems[b])

    def wait_idx(b):
        pltpu.make_async_copy(ei_hbm.at[0, pl.ds(0, MACRO)], sbufs[b],
                              isems[b]).wait()
        pltpu.make_async_copy(ei_hbm.at[1, pl.ds(0, MACRO)], dbufs[b],
                              isems[b]).wait()

    def fire_gathers(b):
        for r in range(CH):
            pltpu.async_copy(table.at[sbufs[b].at[pl.ds(r * BK, BK)]],
                             mbufs[b].at[pl.ds(r * BK, BK), :], gsems[b])

    def drain_gathers(b):
        for r in range(CH):
            pltpu.make_async_copy(table.at[sbufs[b].at[pl.ds(r * BK, BK)]],
                                  mbufs[b].at[pl.ds(r * BK, BK), :],
                                  gsems[b]).wait()

    def fire_scatters(b):
        _copy_rows(cbufs[b], dbufs[b], CH, BK)
        for r in range(CH):
            pltpu.async_copy(mbufs[b].at[pl.ds(r * BK, BK), :],
                             acc.at[cbufs[b].at[r]], ssems[b], add=True)

    def drain_scatters(b):
        for r in range(CH):
            pltpu.make_async_copy(mbufs[b].at[pl.ds(r * BK, BK), :],
                                  acc.at[cbufs[b].at[r]], ssems[b]).wait()

    load_idx(0, 0)
    wait_idx(0)
    fire_gathers(0)
    load_idx(1, 1)

    def body(i, _):
        j = 2 * i

        @pl.when(j > 0)
        def _():
            drain_scatters(1)

        wait_idx(1)
        fire_gathers(1)
        drain_gathers(0)
        fire_scatters(0)
        load_idx(j + 2, 0)
        wait_idx(0)
        drain_gathers(1)
        fire_scatters(1)
        drain_scatters(0)
        fire_gathers(0)

        @pl.when(j + 3 < FULL_MACROS + 1)
        def _():
            load_idx(j + 3, 1)

        return 0

    lax.fori_loop(0, FULL_MACROS // 2, body, 0)
    drain_scatters(1)
    drain_gathers(0)
    fire_scatters(0)
    drain_scatters(0)
    pltpu.sync_copy(ei_hbm.at[0, pl.ds(base + TAIL0, 2 * BK)],
                    s1.at[pl.ds(0, 2 * BK)])
    pltpu.sync_copy(ei_hbm.at[1, pl.ds(base + TAIL0, 2 * BK)],
                    d1.at[pl.ds(0, 2 * BK)])
    for r in range(2):
        pltpu.async_copy(table.at[s1.at[pl.ds(r * BK, BK)]],
                         m1.at[pl.ds(r * BK, BK), :], sg1)
    _copy_rows(c1, d1, 2, BK)
    for r in range(2):
        pltpu.make_async_copy(table.at[s1.at[pl.ds(r * BK, BK)]],
                              m1.at[pl.ds(r * BK, BK), :], sg1).wait()
        pltpu.async_copy(m1.at[pl.ds(r * BK, BK), :],
                         acc.at[c1.at[r]], ss1, add=True)
    pltpu.sync_copy(ei_hbm.at[0, pl.ds(base + TAIL1, TBK)],
                    s0.at[pl.ds(0, TBK)])
    pltpu.sync_copy(ei_hbm.at[1, pl.ds(base + TAIL1, TBK)], ct)
    pltpu.async_copy(table.at[s0.at[pl.ds(0, TBK)]],
                     m0.at[pl.ds(0, TBK), :], sg0)
    pltpu.make_async_copy(table.at[s0.at[pl.ds(0, TBK)]],
                          m0.at[pl.ds(0, TBK), :], sg0).wait()
    pltpu.async_copy(m0.at[pl.ds(0, TBK), :], acc.at[ct], ss1, add=True)
    for r in range(2):
        pltpu.make_async_copy(m1.at[pl.ds(r * BK, BK), :],
                              acc.at[c1.at[r]], ss1).wait()
    pltpu.make_async_copy(m0.at[pl.ds(0, TBK), :], acc.at[ct], ss1).wait()

    plsc.subcore_barrier()
    pltpu.sync_copy(acc.at[pl.ds(s * ROWS_ACC, ROWS_ACC), :],
                    parts.at[c, pl.ds(s * ROWS_ACC, ROWS_ACC), :])


def _k1_body(x_ref, w_ref, dg_ref, xwp_ref, dis_ref):
    dis = lax.rsqrt(dg_ref[0, :, :] + dg_ref[1, :, :] + 1.0)
    xw = jnp.dot(x_ref[...], w_ref[...], preferred_element_type=jnp.float32)
    xwp_ref[...] = xw * dis
    dis_ref[...] = dis


def _k2_body(p_ref, xwp_ref, dis_ref, b1_ref, out_ref):
    dis = dis_ref[...]
    ssum = p_ref[0, :, :] + p_ref[1, :, :] + xwp_ref[...]
    h = jnp.maximum(ssum * dis + b1_ref[...], 0.0)
    out_ref[...] = h * dis


def _k3_body(q_ref, hp_ref, dis_ref, w2_ref, b2_ref, out_ref):
    ssum = (q_ref[0, :, :] + q_ref[1, :, :] + hp_ref[...]) * dis_ref[...]
    y = jnp.dot(ssum, w2_ref[...], preferred_element_type=jnp.float32)
    out_ref[...] = y + b2_ref[...]


def kernel(x, edge_index, W1, b1, W2, b2):
    xp = x.reshape(N // 8, 8 * D_IN)
    w1k = jnp.kron(jnp.eye(8, dtype=jnp.float32), W1)
    w2k = jnp.kron(jnp.eye(8, dtype=jnp.float32), W2)
    b1t = jnp.tile(b1, 8).reshape(1, 8 * D_HID)
    b2t = jnp.tile(b2, 8).reshape(1, 8 * D_OUT)

    deg_parts = _deg_kernel(edge_index)
    dg = deg_parts.reshape(NC, PR, 128)

    xwp, dis = pl.pallas_call(
        _k1_body,
        grid=(GRID,),
        in_specs=[
            pl.BlockSpec((BLK, 8 * D_IN), lambda i: (i, 0)),
            pl.BlockSpec((8 * D_IN, 128), lambda i: (0, 0)),
            pl.BlockSpec((NC, BLK, 128), lambda i: (0, i, 0)),
        ],
        out_specs=[
            pl.BlockSpec((BLK, 128), lambda i: (i, 0)),
            pl.BlockSpec((BLK, 128), lambda i: (i, 0)),
        ],
        out_shape=[
            jax.ShapeDtypeStruct((PR, 128), jnp.float32),
            jax.ShapeDtypeStruct((PR, 128), jnp.float32),
        ],
    )(xp, w1k, dg)

    parts = _agg_kernel(edge_index, xwp.reshape(NACC, D_HID))

    hp = pl.pallas_call(
        _k2_body,
        grid=(GRID,),
        in_specs=[
            pl.BlockSpec((NC, BLK, 128), lambda i: (0, i, 0)),
            pl.BlockSpec((BLK, 128), lambda i: (i, 0)),
            pl.BlockSpec((BLK, 128), lambda i: (i, 0)),
            pl.BlockSpec((1, 128), lambda i: (0, 0)),
        ],
        out_specs=pl.BlockSpec((BLK, 128), lambda i: (i, 0)),
        out_shape=jax.ShapeDtypeStruct((PR, 128), jnp.float32),
    )(parts.reshape(NC, PR, 128), xwp, dis, b1t)

    qarts = _agg_kernel(edge_index, hp.reshape(NACC, D_HID))

    out = pl.pallas_call(
        _k3_body,
        grid=(GRID,),
        in_specs=[
            pl.BlockSpec((NC, BLK, 128), lambda i: (0, i, 0)),
            pl.BlockSpec((BLK, 128), lambda i: (i, 0)),
            pl.BlockSpec((BLK, 128), lambda i: (i, 0)),
            pl.BlockSpec((128, 64), lambda i: (0, 0)),
            pl.BlockSpec((1, 64), lambda i: (0, 0)),
        ],
        out_specs=pl.BlockSpec((BLK, 64), lambda i: (i, 0)),
        out_shape=jax.ShapeDtypeStruct((PR, 64), jnp.float32),
    )(qarts.reshape(NC, PR, 128), hp, dis, w2k, b2t)

    return out.reshape(NACC, D_OUT)[:N]
